# baseline (device time: 127273 ns/iter reference)
import jax
import jax.numpy as jnp
from jax import lax
from jax.experimental import pallas as pl
from jax.experimental.pallas import tpu as pltpu

N_DEV = 4
SQ = 1024
HQ = 8
DH = 128
NR = 4
RQ = SQ // NR
D_MODEL = 1024
SCALE = 0.08838834764831843


def _body(x_ref, wq_ref, wo_ref, k_ref, v_ref, out_ref,
          commq_ref, commo_ref, sA, rA, sB, rB):
    my = lax.axis_index("i")
    left = lax.rem(my + N_DEV - 1, N_DEV)
    right = lax.rem(my + 1, N_DEV)

    barrier = pltpu.get_barrier_semaphore()
    for nbr in (left, right):
        pl.semaphore_signal(barrier, inc=1, device_id=(nbr,),
                            device_id_type=pl.DeviceIdType.MESH)
    pl.semaphore_wait(barrier, 2)

    def copy(src, dst, ss, rs, dev):
        return pltpu.make_async_remote_copy(
            src_ref=src, dst_ref=dst, send_sem=ss, recv_sem=rs,
            device_id=(dev,), device_id_type=pl.DeviceIdType.MESH)

    a_rq = copy(wq_ref, commq_ref.at[0], sA.at[0], rA.at[0], right)
    a_ro = copy(wo_ref, commo_ref.at[0], sA.at[1], rA.at[1], right)
    a_lq = copy(wq_ref, commq_ref.at[1], sA.at[2], rA.at[2], left)
    a_lo = copy(wo_ref, commo_ref.at[1], sA.at[3], rA.at[3], left)
    for r_ in (a_rq, a_ro, a_lq, a_lo):
        r_.start()

    xrs = [x_ref[:, r].reshape(RQ, D_MODEL) for r in range(NR)]

    def accum_group(wq_g, wo_g, g, first):
        hb = g * HQ
        for r in range(NR):
            qt = lax.dot_general(
                wq_g, xrs[r], (((1,), (1,)), ((), ())),
                preferred_element_type=jnp.float32).astype(jnp.bfloat16)
            qt = qt.reshape(HQ, DH, RQ)
            kg = k_ref[r, pl.ds(hb, HQ)].reshape(HQ, RQ, DH)
            vg = v_ref[r, pl.ds(hb, HQ)].reshape(HQ, RQ, DH)
            st = lax.dot_general(
                kg, qt, (((2,), (1,)), ((0,), (0,))),
                preferred_element_type=jnp.float32)
            e32 = jnp.exp(st)
            denom = jnp.sum(e32, axis=1, keepdims=True)
            e = e32.astype(jnp.bfloat16)
            ct = lax.dot_general(
                vg, e, (((1,), (1,)), ((0,), (0,))),
                preferred_element_type=jnp.float32)
            ct = (ct * (1.0 / denom)).astype(jnp.bfloat16)
            o = lax.dot_general(
                ct.reshape(HQ * DH, RQ), wo_g, (((0,), (0,)), ((), ())),
                preferred_element_type=jnp.float32)
            ob = o.reshape(NR, 64, D_MODEL)
            if first:
                out_ref[:, r] = ob
            else:
                out_ref[:, r] += ob

    accum_group(wq_ref[...], wo_ref[...], my, first=True)

    a_rq.wait_recv()
    a_ro.wait_recv()
    b_rq = copy(commq_ref.at[0, 0:2], commq_ref.at[2, 0:2],
                sB.at[0], rB.at[0], right)
    b_ro = copy(commo_ref.at[0, 0:D_MODEL // 2], commo_ref.at[2, 0:D_MODEL // 2],
                sB.at[1], rB.at[1], right)
    b_rq.start()
    b_ro.start()
    accum_group(commq_ref[0], commo_ref[0], left, first=False)

    a_lq.wait_recv()
    a_lo.wait_recv()
    b_lq = copy(commq_ref.at[1, 2:4], commq_ref.at[2, 2:4],
                sB.at[2], rB.at[2], left)
    b_lo = copy(commo_ref.at[1, D_MODEL // 2:D_MODEL],
                commo_ref.at[2, D_MODEL // 2:D_MODEL],
                sB.at[3], rB.at[3], left)
    b_lq.start()
    b_lo.start()
    accum_group(commq_ref[1], commo_ref[1], right, first=False)

    for r_ in (b_rq, b_ro, b_lq, b_lo):
        r_.wait_recv()
    accum_group(commq_ref[2], commo_ref[2], lax.rem(my + 2, N_DEV),
                first=False)

    for r_ in (a_rq, a_ro, a_lq, a_lo, b_rq, b_ro, b_lq, b_lo):
        r_.wait_send()


def kernel(x, Wq, K_ext, V_ext, Wo):
    x_b = x[0].reshape(NR, NR, 64, D_MODEL).astype(jnp.bfloat16)
    wq_p = ((Wq * SCALE).astype(jnp.bfloat16)
            .reshape(D_MODEL, HQ // 2, 2 * DH).transpose(1, 0, 2))
    wo_b = Wo.astype(jnp.bfloat16)
    k_t = (K_ext[0].reshape(NR, NR, 64, N_DEV * HQ, DH)
           .transpose(1, 3, 0, 2, 4).astype(jnp.bfloat16))
    v_t = (V_ext[0].reshape(NR, NR, 64, N_DEV * HQ, DH)
           .transpose(1, 3, 0, 2, 4).astype(jnp.bfloat16))

    out = pl.pallas_call(
        _body,
        out_shape=jax.ShapeDtypeStruct((NR, NR, 64, D_MODEL), jnp.float32),
        in_specs=[pl.BlockSpec(memory_space=pltpu.VMEM)] * 5,
        out_specs=pl.BlockSpec(memory_space=pltpu.VMEM),
        scratch_shapes=[
            pltpu.VMEM((3, HQ // 2, D_MODEL, 2 * DH), jnp.bfloat16),
            pltpu.VMEM((3, D_MODEL, D_MODEL), jnp.bfloat16),
            pltpu.SemaphoreType.DMA((4,)),
            pltpu.SemaphoreType.DMA((4,)),
            pltpu.SemaphoreType.DMA((4,)),
            pltpu.SemaphoreType.DMA((4,)),
        ],
        compiler_params=pltpu.CompilerParams(collective_id=0),
    )(x_b, wq_p, wo_b, k_t, v_t)

    return out.reshape(1, SQ, D_MODEL)


# device time: 117005 ns/iter; 1.0878x vs baseline; 1.0878x over previous
import jax
import jax.numpy as jnp
from jax import lax
from jax.experimental import pallas as pl
from jax.experimental.pallas import tpu as pltpu

N_DEV = 4
SQ = 1024
HQ = 8
DH = 128
NR = 4
RQ = SQ // NR
D_MODEL = 1024
SCALE = 0.08838834764831843
HALF = D_MODEL // 2


def _body(x_ref, wq_ref, wo_ref, k_ref, v_ref, out_ref,
          commq_ref, commo_ref, sA, rA, sB, rB):
    my = lax.axis_index("i")
    left = lax.rem(my + N_DEV - 1, N_DEV)
    right = lax.rem(my + 1, N_DEV)

    barrier = pltpu.get_barrier_semaphore()
    for nbr in (left, right):
        pl.semaphore_signal(barrier, inc=1, device_id=(nbr,),
                            device_id_type=pl.DeviceIdType.MESH)
    pl.semaphore_wait(barrier, 2)

    def copy(src, dst, ss, rs, dev):
        return pltpu.make_async_remote_copy(
            src_ref=src, dst_ref=dst, send_sem=ss, recv_sem=rs,
            device_id=(dev,), device_id_type=pl.DeviceIdType.MESH)

    def hopA(dev, slot, base):
        return (
            copy(wq_ref.at[0:2], commq_ref.at[slot, 0:2],
                 sA.at[base + 0], rA.at[base + 0], dev),
            copy(wo_ref.at[0:HALF], commo_ref.at[slot, 0:HALF],
                 sA.at[base + 1], rA.at[base + 1], dev),
            copy(wq_ref.at[2:4], commq_ref.at[slot, 2:4],
                 sA.at[base + 2], rA.at[base + 2], dev),
            copy(wo_ref.at[HALF:D_MODEL], commo_ref.at[slot, HALF:D_MODEL],
                 sA.at[base + 3], rA.at[base + 3], dev),
        )

    a_r = hopA(right, 0, 0)
    a_l = hopA(left, 1, 4)
    for r_ in (a_r[0], a_r[1], a_l[0], a_l[1], a_r[2], a_r[3], a_l[2], a_l[3]):
        r_.start()

    xrs = [x_ref[:, r].reshape(RQ, D_MODEL).astype(jnp.bfloat16)
           for r in range(NR)]

    def accum_half(wq_h, wo_h, hb, first=False):
        nh = HQ // 2
        for r in range(NR):
            qt = lax.dot_general(
                wq_h, xrs[r], (((1,), (1,)), ((), ())),
                preferred_element_type=jnp.float32).astype(jnp.bfloat16)
            qt = qt.reshape(nh, DH, RQ)
            kg = k_ref[r, pl.ds(hb, nh)].reshape(nh, RQ, DH)
            vg = v_ref[r, pl.ds(hb, nh)].reshape(nh, RQ, DH)
            st = lax.dot_general(
                kg, qt, (((2,), (1,)), ((0,), (0,))),
                preferred_element_type=jnp.float32)
            e32 = jnp.exp(st)
            denom = jnp.sum(e32, axis=1, keepdims=True)
            e = e32.astype(jnp.bfloat16)
            ct = lax.dot_general(
                vg, e, (((1,), (1,)), ((0,), (0,))),
                preferred_element_type=jnp.float32)
            ct = (ct * (1.0 / denom)).astype(jnp.bfloat16)
            o = lax.dot_general(
                ct.reshape(nh * DH, RQ), wo_h, (((0,), (0,)), ((), ())),
                preferred_element_type=jnp.float32)
            ob = o.reshape(NR, 64, D_MODEL).astype(jnp.bfloat16)
            if first:
                out_ref[:, r] = ob
            else:
                out_ref[:, r] += ob

    accum_half(wq_ref[0:2], wo_ref[0:HALF], my * HQ, first=True)
    accum_half(wq_ref[2:4], wo_ref[HALF:D_MODEL], my * HQ + HQ // 2)

    a_r[0].wait_recv()
    a_r[1].wait_recv()
    b_r = (copy(commq_ref.at[0, 0:2], commq_ref.at[2, 0:2],
                sB.at[0], rB.at[0], right),
           copy(commo_ref.at[0, 0:HALF], commo_ref.at[2, 0:HALF],
                sB.at[1], rB.at[1], right))
    b_r[0].start()
    b_r[1].start()
    accum_half(commq_ref[0, 0:2], commo_ref[0, 0:HALF], left * HQ)

    a_l[0].wait_recv()
    a_l[1].wait_recv()
    accum_half(commq_ref[1, 0:2], commo_ref[1, 0:HALF], right * HQ)

    a_l[2].wait_recv()
    a_l[3].wait_recv()
    b_l = (copy(commq_ref.at[1, 2:4], commq_ref.at[2, 2:4],
                sB.at[2], rB.at[2], left),
           copy(commo_ref.at[1, HALF:D_MODEL], commo_ref.at[2, HALF:D_MODEL],
                sB.at[3], rB.at[3], left))
    b_l[0].start()
    b_l[1].start()
    accum_half(commq_ref[1, 2:4], commo_ref[1, HALF:D_MODEL],
               right * HQ + HQ // 2)

    a_r[2].wait_recv()
    a_r[3].wait_recv()
    accum_half(commq_ref[0, 2:4], commo_ref[0, HALF:D_MODEL],
               left * HQ + HQ // 2)

    diag = lax.rem(my + 2, N_DEV)
    b_r[0].wait_recv()
    b_r[1].wait_recv()
    accum_half(commq_ref[2, 0:2], commo_ref[2, 0:HALF], diag * HQ)
    b_l[0].wait_recv()
    b_l[1].wait_recv()
    accum_half(commq_ref[2, 2:4], commo_ref[2, HALF:D_MODEL],
               diag * HQ + HQ // 2)

    for r_ in (*a_r, *a_l, *b_r, *b_l):
        r_.wait_send()


def kernel(x, Wq, K_ext, V_ext, Wo):
    x_b = x[0].reshape(NR, NR, 64, D_MODEL)
    wq_p = ((Wq * SCALE).astype(jnp.bfloat16)
            .reshape(D_MODEL, HQ // 2, 2 * DH).transpose(1, 0, 2))
    wo_b = Wo.astype(jnp.bfloat16)
    k_t = (K_ext[0].reshape(NR, NR, 64, N_DEV * HQ, DH)
           .transpose(1, 3, 0, 2, 4).astype(jnp.bfloat16))
    v_t = (V_ext[0].reshape(NR, NR, 64, N_DEV * HQ, DH)
           .transpose(1, 3, 0, 2, 4).astype(jnp.bfloat16))

    out = pl.pallas_call(
        _body,
        out_shape=jax.ShapeDtypeStruct((NR, NR, 64, D_MODEL), jnp.bfloat16),
        in_specs=[pl.BlockSpec(memory_space=pltpu.VMEM)] * 5,
        out_specs=pl.BlockSpec(memory_space=pltpu.VMEM),
        scratch_shapes=[
            pltpu.VMEM((3, HQ // 2, D_MODEL, 2 * DH), jnp.bfloat16),
            pltpu.VMEM((3, D_MODEL, D_MODEL), jnp.bfloat16),
            pltpu.SemaphoreType.DMA((8,)),
            pltpu.SemaphoreType.DMA((8,)),
            pltpu.SemaphoreType.DMA((4,)),
            pltpu.SemaphoreType.DMA((4,)),
        ],
        compiler_params=pltpu.CompilerParams(collective_id=0),
    )(x_b, wq_p, wo_b, k_t, v_t)

    return out.reshape(1, SQ, D_MODEL)
